# baseline (speedup 1.0000x reference)
import jax
import jax.numpy as jnp
from jax.experimental import pallas as pl

N = 50000
NGRAPH = 512
NRBF = 8
CUTOFF = 5.0


def _mlp_kernel(h_ref, p1_ref, b1_ref, p2_ref, b2_ref, out_ref):
    h = h_ref[...]
    z = h @ p1_ref[...] + b1_ref[...]
    z = z * jax.nn.sigmoid(z)
    out_ref[...] = z @ p2_ref[...] + b2_ref[...]


def _post_mlp(h, P1, b1, P2, b2):
    B = 2000
    grid = (N // B,)
    return pl.pallas_call(
        _mlp_kernel,
        grid=grid,
        in_specs=[
            pl.BlockSpec((B, 64), lambda i: (i, 0)),
            pl.BlockSpec((64, 64), lambda i: (0, 0)),
            pl.BlockSpec((1, 64), lambda i: (0, 0)),
            pl.BlockSpec((64, 1), lambda i: (0, 0)),
            pl.BlockSpec((1, 1), lambda i: (0, 0)),
        ],
        out_specs=pl.BlockSpec((B, 1), lambda i: (i, 0)),
        out_shape=jax.ShapeDtypeStruct((N, 1), jnp.float32),
    )(h, P1, b1.reshape(1, 64), P2, b2.reshape(1, 1))


def kernel(x, pos, edge_index, period_vec, batch, elem_table, W_embed, W1,
           Wrbf, Wsh, W2, Wattr, Wself, P1, b1, P2, b2):
    def silu(v):
        return v * jax.nn.sigmoid(v)

    x_attr = jnp.take(elem_table, x[:, 0], axis=0)
    h = x_attr @ W_embed
    src = edge_index[0]
    dst = edge_index[1]
    edge_vec = jnp.take(pos, dst, axis=0) - jnp.take(pos, src, axis=0) + period_vec
    lengths = jnp.sqrt(jnp.sum(edge_vec ** 2, axis=-1) + 1e-12)
    unit = edge_vec / lengths[:, None]
    edge_sh = jnp.concatenate(
        [jnp.ones((edge_vec.shape[0], 1), dtype=edge_vec.dtype), unit], axis=-1)
    n = jnp.arange(1, NRBF + 1, dtype=jnp.float32)
    env = 0.5 * (jnp.cos(jnp.pi * jnp.clip(lengths / CUTOFF, 0.0, 1.0)) + 1.0)
    rbf = jnp.sin(n[None, :] * jnp.pi * lengths[:, None] / CUTOFF) / lengths[:, None] * env[:, None]
    for i in range(3):
        m = (jnp.take(h, src, axis=0) @ W1[i]) * (rbf @ Wrbf[i]) * (edge_sh @ Wsh[i])
        agg = jax.ops.segment_sum(m, dst, num_segments=N)
        h = silu(agg @ W2[i] + x_attr @ Wattr[i] + h @ Wself[i])
    hs = _post_mlp(h, P1, b1, P2, b2)[:, 0]
    e = jax.ops.segment_sum(hs, batch, num_segments=NGRAPH)
    return e[:, None]


# speedup vs baseline: 1.1322x; 1.1322x over previous
import functools
import jax
import jax.numpy as jnp
from jax import lax
from jax.experimental import pallas as pl
from jax.experimental.pallas import tpu as pltpu
from jax.experimental.pallas import tpu_sc as plsc

N = 50000
E = 800000
NGRAPH = 512
NRBF = 8
CUTOFF = 5.0
HID = 64
ATTR = 16

NP = 53248
EP = 819200
NROWS = NP // 128
EROWS = EP // 128
NPH = NP // 2
ACC = NPH + 128
ZSL = ACC // 16
WSL = NPH // 16

_mesh = plsc.VectorSubcoreMesh(core_axis_name="c", subcore_axis_name="s")
_sc_params = pltpu.CompilerParams(use_tc_tiling_on_sc=False)


@functools.partial(
    pl.kernel, mesh=_mesh,
    out_type=(
        jax.ShapeDtypeStruct((NROWS, 128, ATTR), jnp.float32),
        jax.ShapeDtypeStruct((EROWS, 128, 16), jnp.float32),
        jax.ShapeDtypeStruct((EROWS, 128, 16), jnp.float32),
    ),
    scratch_types=[
        pltpu.VMEM((8, 128), jnp.int32),
        pltpu.VMEM((8, 128, ATTR), jnp.float32),
        pltpu.VMEM((8, 128), jnp.int32),
        pltpu.VMEM((8, 128, 16), jnp.float32),
        pltpu.SemaphoreType.DMA,
    ],
    compiler_params=_sc_params,
)
def _sc_prep(elem_hbm, pos_hbm, xp_hbm, src_hbm, dst_hbm,
             xattr_hbm, psrc_hbm, pdst_hbm,
             nidx, xrows, eidx, prow, sem):
    c = lax.axis_index("c")
    s = lax.axis_index("s")
    w = s * 2 + c
    nch = NROWS // 8

    def nchunk(k, carry):
        cid = w + 32 * k

        @pl.when(cid < nch)
        def _():
            r0 = cid * 8
            pltpu.sync_copy(xp_hbm.at[pl.ds(r0, 8)], nidx)
            cps = [pltpu.async_copy(elem_hbm.at[nidx.at[r]], xrows.at[r], sem)
                   for r in range(8)]
            for cp in cps:
                cp.wait()
            pltpu.sync_copy(xrows, xattr_hbm.at[pl.ds(r0, 8)])

        return carry

    lax.fori_loop(0, (nch + 31) // 32, nchunk, 0)

    def chunk(k, carry):
        rr = w * (EROWS // 32) + k * 8
        pltpu.sync_copy(src_hbm.at[pl.ds(rr, 8)], eidx)
        g = [pltpu.async_copy(pos_hbm.at[eidx.at[r]], prow.at[r], sem)
             for r in range(8)]
        for cp in g:
            cp.wait()
        pltpu.sync_copy(prow, psrc_hbm.at[pl.ds(rr, 8)])
        pltpu.sync_copy(dst_hbm.at[pl.ds(rr, 8)], eidx)
        g = [pltpu.async_copy(pos_hbm.at[eidx.at[r]], prow.at[r], sem)
             for r in range(8)]
        for cp in g:
            cp.wait()
        pltpu.sync_copy(prow, pdst_hbm.at[pl.ds(rr, 8)])
        return carry

    lax.fori_loop(0, (EROWS // 32) // 8, chunk, 0)


@functools.partial(
    pl.kernel, mesh=_mesh,
    out_type=jax.ShapeDtypeStruct((2, NP, 32), jnp.float32),
    scratch_types=[
        pltpu.VMEM((8, 128), jnp.int32),
        pltpu.VMEM((8, 128), jnp.int32),
        pltpu.VMEM((8, 128), jnp.int32),
        pltpu.VMEM((8, 128, 32), jnp.float32),
        pltpu.VMEM((8, 128, 32), jnp.float32),
        pltpu.VMEM_SHARED((ACC, 32), jnp.float32),
        pltpu.SemaphoreType.DMA,
    ],
    compiler_params=_sc_params,
)
def _sc_edge(hw_hbm, f_hbm, src_hbm, dst_hbm, zeros_hbm, out_hbm,
             sidx, didx, didx2, rows, fbuf, acc, sem):
    c = lax.axis_index("c")
    s = lax.axis_index("s")
    rpw = EROWS // 16

    def do_half(cc):
        tab = hw_hbm.at[cc]
        f4 = f_hbm.at[cc]
        for p in range(2):
            base = p * NPH
            pltpu.sync_copy(zeros_hbm.at[pl.ds(s * ZSL, ZSL)],
                            acc.at[pl.ds(s * ZSL, ZSL)])
            plsc.subcore_barrier()

            def chunk(k, carry):
                r0 = s * rpw + k * 8
                pltpu.sync_copy(src_hbm.at[pl.ds(r0, 8)], sidx)
                pltpu.sync_copy(dst_hbm.at[pl.ds(r0, 8)], didx)
                pltpu.sync_copy(f4.at[pl.ds(r0, 8)], fbuf)
                g = [pltpu.async_copy(tab.at[sidx.at[r]], rows.at[r], sem)
                     for r in range(8)]
                for r in range(8):
                    for v in range(8):
                        sl = pl.ds(v * 16, 16)
                        d = didx[r, sl] - base
                        ok = (d >= 0) & (d < NPH)
                        didx2[r, sl] = jnp.where(ok, d, NPH)
                for cp in g:
                    cp.wait()

                def mulrow(j, cy):
                    for r in range(8):
                        for v in range(2):
                            sl = pl.ds(v * 16, 16)
                            rows[r, j, sl] = rows[r, j, sl] * fbuf[r, j, sl]
                    return cy

                lax.fori_loop(0, 128, mulrow, 0)
                for r in range(8):
                    pltpu.sync_copy(rows.at[r], acc.at[didx2.at[r]], add=True)
                return carry

            lax.fori_loop(0, rpw // 8, chunk, 0)
            plsc.subcore_barrier()
            pltpu.sync_copy(acc.at[pl.ds(s * WSL, WSL)],
                            out_hbm.at[cc].at[pl.ds(base + s * WSL, WSL)])
            plsc.subcore_barrier()

    @pl.when(c == 0)
    def _():
        do_half(0)

    @pl.when(c == 1)
    def _():
        do_half(1)


_CE = 3200
_CN = 3328


def _geo_body(ps_ref, pd_ref, pv_ref, g_ref):
    i = pl.program_id(0)
    ev = pd_ref[...] - ps_ref[...] + pv_ref[...]
    len2 = jnp.sum(ev * ev, axis=1, keepdims=True) + 1e-12
    lengths = jnp.sqrt(len2)
    inv = 1.0 / lengths
    env = 0.5 * (jnp.cos(jnp.pi * jnp.clip(lengths / CUTOFF, 0.0, 1.0)) + 1.0)
    nvec = lax.broadcasted_iota(jnp.int32, (1, NRBF), 1).astype(jnp.float32) + 1.0
    rbf = jnp.sin(lengths * nvec * (jnp.pi / CUTOFF)) * inv * env
    unit = ev[:, 0:3] * inv
    ones = jnp.ones((_CE, 1), jnp.float32)
    zer = jnp.zeros((_CE, 4), jnp.float32)
    g = jnp.concatenate([rbf, ones, unit, zer], axis=1)
    rid = i * _CE + lax.broadcasted_iota(jnp.int32, (_CE, 1), 0)
    g_ref[...] = jnp.where(rid < E, g, 0.0)


def _tc_geo(psrc, pdst, per16):
    return pl.pallas_call(
        _geo_body,
        grid=(EP // _CE,),
        in_specs=[pl.BlockSpec((_CE, 16), lambda i: (i, 0))] * 3,
        out_specs=pl.BlockSpec((_CE, 16), lambda i: (i, 0)),
        out_shape=jax.ShapeDtypeStruct((EP, 16), jnp.float32),
    )(psrc, pdst, per16)


def _f_body(g_ref, wr_ref, ws_ref, f_ref):
    g = g_ref[...]
    f = (jnp.dot(g[:, 0:NRBF], wr_ref[...],
                 preferred_element_type=jnp.float32) *
         jnp.dot(g[:, NRBF:NRBF + 4], ws_ref[...],
                 preferred_element_type=jnp.float32))
    f_ref[0, ...] = f[:, 0:32]
    f_ref[1, ...] = f[:, 32:64]


def _tc_factor(g, wrbf, wsh):
    return pl.pallas_call(
        _f_body,
        grid=(EP // _CE,),
        in_specs=[
            pl.BlockSpec((_CE, 16), lambda i: (i, 0)),
            pl.BlockSpec((NRBF, HID), lambda i: (0, 0)),
            pl.BlockSpec((4, HID), lambda i: (0, 0)),
        ],
        out_specs=pl.BlockSpec((2, _CE, 32), lambda i: (0, i, 0)),
        out_shape=jax.ShapeDtypeStruct((2, EP, 32), jnp.float32),
    )(g, wrbf, wsh)


def _node0_body(xa_ref, we_ref, w1_ref, h_ref, hw_ref):
    h = jnp.dot(xa_ref[...], we_ref[...], preferred_element_type=jnp.float32)
    h_ref[...] = h
    hw = jnp.dot(h, w1_ref[...], preferred_element_type=jnp.float32)
    hw_ref[0, ...] = hw[:, 0:32]
    hw_ref[1, ...] = hw[:, 32:64]


def _tc_node0(x_attr, w_embed, w1):
    return pl.pallas_call(
        _node0_body,
        grid=(NP // _CN,),
        in_specs=[
            pl.BlockSpec((_CN, ATTR), lambda i: (i, 0)),
            pl.BlockSpec((ATTR, HID), lambda i: (0, 0)),
            pl.BlockSpec((HID, HID), lambda i: (0, 0)),
        ],
        out_specs=[
            pl.BlockSpec((_CN, HID), lambda i: (i, 0)),
            pl.BlockSpec((2, _CN, 32), lambda i: (0, i, 0)),
        ],
        out_shape=[
            jax.ShapeDtypeStruct((NP, HID), jnp.float32),
            jax.ShapeDtypeStruct((2, NP, 32), jnp.float32),
        ],
    )(x_attr, w_embed, w1)


def _silu(v):
    return v * jax.nn.sigmoid(v)


def _upd_body(agg_ref, xa_ref, h_ref, w2_ref, wa_ref, wf_ref, w1n_ref,
              hn_ref, hw_ref):
    agg = jnp.concatenate([agg_ref[0, ...], agg_ref[1, ...]], axis=1)
    u = _silu(jnp.dot(agg, w2_ref[...], preferred_element_type=jnp.float32)
              + jnp.dot(xa_ref[...], wa_ref[...],
                        preferred_element_type=jnp.float32)
              + jnp.dot(h_ref[...], wf_ref[...],
                        preferred_element_type=jnp.float32))
    hn_ref[...] = u
    hw = jnp.dot(u, w1n_ref[...], preferred_element_type=jnp.float32)
    hw_ref[0, ...] = hw[:, 0:32]
    hw_ref[1, ...] = hw[:, 32:64]


def _tc_update(agg, x_attr, h, w2, wattr, wself, w1n):
    return pl.pallas_call(
        _upd_body,
        grid=(NP // _CN,),
        in_specs=[
            pl.BlockSpec((2, _CN, 32), lambda i: (0, i, 0)),
            pl.BlockSpec((_CN, ATTR), lambda i: (i, 0)),
            pl.BlockSpec((_CN, HID), lambda i: (i, 0)),
            pl.BlockSpec((HID, HID), lambda i: (0, 0)),
            pl.BlockSpec((ATTR, HID), lambda i: (0, 0)),
            pl.BlockSpec((HID, HID), lambda i: (0, 0)),
            pl.BlockSpec((HID, HID), lambda i: (0, 0)),
        ],
        out_specs=[
            pl.BlockSpec((_CN, HID), lambda i: (i, 0)),
            pl.BlockSpec((2, _CN, 32), lambda i: (0, i, 0)),
        ],
        out_shape=[
            jax.ShapeDtypeStruct((NP, HID), jnp.float32),
            jax.ShapeDtypeStruct((2, NP, 32), jnp.float32),
        ],
    )(agg, x_attr, h, w2, wattr, wself, w1n)


def _final_body(agg_ref, xa_ref, h_ref, w2_ref, wa_ref, wf_ref,
                p1_ref, b1_ref, p2_ref, b2_ref, bid_ref, e_ref):
    i = pl.program_id(0)
    agg = jnp.concatenate([agg_ref[0, ...], agg_ref[1, ...]], axis=1)
    u = _silu(jnp.dot(agg, w2_ref[...], preferred_element_type=jnp.float32)
              + jnp.dot(xa_ref[...], wa_ref[...],
                        preferred_element_type=jnp.float32)
              + jnp.dot(h_ref[...], wf_ref[...],
                        preferred_element_type=jnp.float32))
    z = _silu(jnp.dot(u, p1_ref[...], preferred_element_type=jnp.float32)
              + b1_ref[...])
    hs = jnp.dot(z, p2_ref[...], preferred_element_type=jnp.float32) + b2_ref[...]
    gcol = lax.broadcasted_iota(jnp.int32, (_CN, NGRAPH), 1)
    oh = (gcol == bid_ref[...]).astype(jnp.float32)
    eblk = lax.dot_general(oh, hs, (((0,), (0,)), ((), ())),
                           preferred_element_type=jnp.float32)

    @pl.when(i == 0)
    def _():
        e_ref[...] = jnp.zeros_like(e_ref)

    e_ref[...] += eblk


def _tc_final(agg, x_attr, h, w2, wattr, wself, p1, b1, p2, b2, bid):
    return pl.pallas_call(
        _final_body,
        grid=(NP // _CN,),
        in_specs=[
            pl.BlockSpec((2, _CN, 32), lambda i: (0, i, 0)),
            pl.BlockSpec((_CN, ATTR), lambda i: (i, 0)),
            pl.BlockSpec((_CN, HID), lambda i: (i, 0)),
            pl.BlockSpec((HID, HID), lambda i: (0, 0)),
            pl.BlockSpec((ATTR, HID), lambda i: (0, 0)),
            pl.BlockSpec((HID, HID), lambda i: (0, 0)),
            pl.BlockSpec((HID, HID), lambda i: (0, 0)),
            pl.BlockSpec((1, HID), lambda i: (0, 0)),
            pl.BlockSpec((HID, 1), lambda i: (0, 0)),
            pl.BlockSpec((1, 1), lambda i: (0, 0)),
            pl.BlockSpec((_CN, 1), lambda i: (i, 0)),
        ],
        out_specs=pl.BlockSpec((NGRAPH, 1), lambda i: (0, 0)),
        out_shape=jax.ShapeDtypeStruct((NGRAPH, 1), jnp.float32),
    )(agg, x_attr, h, w2, wattr, wself, p1, b1, p2, b2, bid)


def kernel(x, pos, edge_index, period_vec, batch, elem_table, W_embed, W1,
           Wrbf, Wsh, W2, Wattr, Wself, P1, b1, P2, b2):
    f32 = jnp.float32
    i32 = jnp.int32

    xp = jnp.concatenate([x[:, 0].astype(i32),
                          jnp.zeros((NP - N,), i32)]).reshape(NROWS, 128)
    pos16 = jnp.concatenate(
        [jnp.concatenate([pos.astype(f32), jnp.zeros((N, 13), f32)], axis=1),
         jnp.zeros((NP - N, 16), f32)], axis=0)
    src = jnp.concatenate([edge_index[0].astype(i32),
                           jnp.zeros((EP - E,), i32)]).reshape(EROWS, 128)
    dst = jnp.concatenate([edge_index[1].astype(i32),
                           jnp.zeros((EP - E,), i32)]).reshape(EROWS, 128)
    per16 = jnp.concatenate(
        [jnp.concatenate([period_vec.astype(f32), jnp.zeros((E, 13), f32)],
                         axis=1),
         jnp.zeros((EP - E, 16), f32)], axis=0)
    bid = jnp.concatenate([batch.astype(i32),
                           jnp.full((NP - N,), -1, i32)]).reshape(NP, 1)
    znp = jnp.zeros((ACC, 32), f32)

    xattr3, psrc3, pdst3 = _sc_prep(elem_table.astype(f32), pos16, xp, src, dst)
    x_attr = xattr3.reshape(NP, ATTR)
    g = _tc_geo(psrc3.reshape(EP, 16), pdst3.reshape(EP, 16), per16)

    h, hw = _tc_node0(x_attr, W_embed, W1[0])
    for i in range(3):
        f = _tc_factor(g, Wrbf[i], Wsh[i])
        agg = _sc_edge(hw, f.reshape(2, EROWS, 128, 32), src, dst, znp)
        if i < 2:
            h, hw = _tc_update(agg, x_attr, h, W2[i], Wattr[i], Wself[i],
                               W1[i + 1])
        else:
            e = _tc_final(agg, x_attr, h, W2[i], Wattr[i], Wself[i],
                          P1, b1.reshape(1, HID), P2, b2.reshape(1, 1), bid)
    return e
